# baseline (speedup 1.0000x reference)
import functools

import jax
import jax.numpy as jnp
from jax import lax
from jax.experimental import pallas as pl
from jax.experimental.pallas import tpu as pltpu
from jax.experimental.pallas import tpu_sc as plsc

_PP = 8736
_NCHUNK = _PP // 16

_B, _P, _C, _K = 32, 8732, 21, 12
_THRESHOLD = 0.5
_NEG_POS_RATIO = 3
_ALPHA = 1.0


def _sc_match_body(px0_h, py0_h, px1_h, py1_h, areap_h, boxes_h, labels_h,
                   label_out, obj_out,
                   px0_v, py0_v, px1_v, py1_v, areap_v,
                   box_v, lab_v, bo_v, bk_v, lo_v):
    i32 = jnp.int32
    f32 = jnp.float32
    wid = lax.axis_index("s") * 2 + lax.axis_index("c")
    pltpu.sync_copy(px0_h, px0_v)
    pltpu.sync_copy(py0_h, py0_v)
    pltpu.sync_copy(px1_h, px1_v)
    pltpu.sync_copy(py1_h, py1_v)
    pltpu.sync_copy(areap_h, areap_v)
    pltpu.sync_copy(boxes_h.at[pl.ds(wid * 64, 64)], box_v)
    pltpu.sync_copy(labels_h.at[pl.ds(wid * 16, 16)], lab_v)

    lane = lax.iota(i32, 16)
    pfeo = []
    for k in range(_K):
        bv = box_v[pl.ds((k // 4) * 16, 16)]
        j = (k % 4) * 4
        bx0 = lax.broadcast(bv[j + 0], (16,))
        by0 = lax.broadcast(bv[j + 1], (16,))
        bx1 = lax.broadcast(bv[j + 2], (16,))
        by1 = lax.broadcast(bv[j + 3], (16,))
        areab = (bx1 - bx0) * (by1 - by0)

        def body(i, carry, bx0=bx0, by0=by0, bx1=bx1, by1=by1, areab=areab, k=k):
            mkv, mki = carry
            sl = pl.ds(i * 16, 16)
            iw = jnp.maximum(jnp.minimum(bx1, px1_v[sl]) - jnp.maximum(bx0, px0_v[sl]), 0.0)
            ih = jnp.maximum(jnp.minimum(by1, py1_v[sl]) - jnp.maximum(by0, py0_v[sl]), 0.0)
            inter = iw * ih
            ov = inter / (areab + areap_v[sl] - inter)
            if k == 0:
                bo_v[sl] = ov
                bk_v[sl] = jnp.zeros((16,), i32)
            else:
                cur = bo_v[sl]
                curk = bk_v[sl]
                upd = ov > cur
                bo_v[sl] = jnp.where(upd, ov, cur)
                bk_v[sl] = jnp.where(upd, k, curk)
            upd2 = ov > mkv
            mkv = jnp.where(upd2, ov, mkv)
            mki = jnp.where(upd2, i, mki)
            return mkv, mki

        mkv, mki = lax.fori_loop(
            0, _NCHUNK, body,
            (jnp.full((16,), -1.0, f32), jnp.zeros((16,), i32)))
        flat = mki * 16 + lane
        m = mkv[0]
        fi = flat[0]
        for j in range(1, 16):
            vj = mkv[j]
            fj = flat[j]
            take = (vj > m) | ((vj == m) & (fj < fi))
            m = jnp.where(take, vj, m)
            fi = jnp.where(take, fj, fi)
        pfeo.append(fi)

    pfeo_b = [lax.broadcast(p, (16,)) for p in pfeo]
    lv = lab_v[...]
    lab_b = [lax.broadcast(lv[k], (16,)) for k in range(_K)]

    def body3(i, _):
        sl = pl.ds(i * 16, 16)
        bk = bk_v[sl]
        bo = bo_v[sl]
        flat = lax.broadcast(i * 16, (16,)) + lane
        for k in range(_K):
            hit = flat == pfeo_b[k]
            bk = jnp.where(hit, k, bk)
            bo = jnp.where(hit, 1.0, bo)
        lab = jnp.zeros((16,), i32)
        for k in range(_K):
            lab = jnp.where(bk == k, lab_b[k], lab)
        lo_v[sl] = jnp.where(bo < _THRESHOLD, 0, lab)
        bk_v[sl] = bk
        return 0

    lax.fori_loop(0, _NCHUNK, body3, 0)
    pltpu.sync_copy(lo_v, label_out.at[wid])
    pltpu.sync_copy(bk_v, obj_out.at[wid])


def _make_sc_match():
    mesh = plsc.VectorSubcoreMesh(core_axis_name="c", subcore_axis_name="s")
    f32 = jnp.float32
    i32 = jnp.int32
    return functools.partial(
        pl.kernel,
        out_type=[jax.ShapeDtypeStruct((_B, _PP), i32),
                  jax.ShapeDtypeStruct((_B, _PP), i32)],
        mesh=mesh,
        scratch_types=[
            pltpu.VMEM((_PP,), f32), pltpu.VMEM((_PP,), f32),
            pltpu.VMEM((_PP,), f32), pltpu.VMEM((_PP,), f32),
            pltpu.VMEM((_PP,), f32),
            pltpu.VMEM((64,), f32), pltpu.VMEM((16,), i32),
            pltpu.VMEM((_PP,), f32), pltpu.VMEM((_PP,), i32),
            pltpu.VMEM((_PP,), i32),
        ],
    )(_sc_match_body)


def _ce_body(priors_ref, boxes_ref, label_ref, obj_ref,
             locs1_ref, scores1_ref, locs2_ref, scores2_ref,
             cn1_ref, cn2_ref, part_ref):
    f32 = jnp.float32
    K, P, C = _K, _P, _C

    pcx = priors_ref[0:1, :]
    pcy = priors_ref[1:2, :]
    pw = priors_ref[2:3, :]
    ph = priors_ref[3:4, :]

    b = boxes_ref[0]
    bx0 = b[:, 0:1]
    by0 = b[:, 1:2]
    bx1 = b[:, 2:3]
    by1 = b[:, 3:4]

    label = label_ref[0]
    obj = obj_ref[0]
    pos = label > 0
    posf = pos.astype(f32)
    npos = jnp.sum(posf)

    kiota = lax.broadcasted_iota(jnp.int32, (K, P), 0)
    onehotf = (obj == kiota).astype(f32)
    gx0 = jnp.sum(onehotf * bx0, axis=0, keepdims=True)
    gy0 = jnp.sum(onehotf * by0, axis=0, keepdims=True)
    gx1 = jnp.sum(onehotf * bx1, axis=0, keepdims=True)
    gy1 = jnp.sum(onehotf * by1, axis=0, keepdims=True)
    cx = (gx0 + gx1) * 0.5
    cy = (gy0 + gy1) * 0.5
    w = gx1 - gx0
    h = gy1 - gy0
    tl0 = (cx - pcx) / (pw / 10.0)
    tl1 = (cy - pcy) / (ph / 10.0)
    tl2 = jnp.log(w / pw) * 5.0
    tl3 = jnp.log(h / ph) * 5.0

    ciota = lax.broadcasted_iota(jnp.int32, (C, P), 0)

    def branch(locs_ref, scores_ref, cn_ref):
        loc_abs = (jnp.abs(locs_ref[0, 0:1, :] - tl0)
                   + jnp.abs(locs_ref[0, 1:2, :] - tl1)
                   + jnp.abs(locs_ref[0, 2:3, :] - tl2)
                   + jnp.abs(locs_ref[0, 3:4, :] - tl3))
        loc_sum = jnp.sum(loc_abs * posf)
        s = scores_ref[0]
        mx = jnp.max(s, axis=0, keepdims=True)
        lse = jnp.log(jnp.sum(jnp.exp(s - mx), axis=0, keepdims=True)) + mx
        strue = jnp.sum(jnp.where(ciota == label, s, 0.0), axis=0, keepdims=True)
        ce = lse - strue
        cep = jnp.sum(ce * posf)
        cn = jnp.maximum(jnp.where(pos, 0.0, ce), 0.0)
        cn_ref[0] = cn
        return loc_sum, cep

    l1, c1 = branch(locs1_ref, scores1_ref, cn1_ref)
    l2, c2 = branch(locs2_ref, scores2_ref, cn2_ref)

    lane = lax.broadcasted_iota(jnp.int32, (1, 128), 1)
    row = (npos * (lane == 0).astype(f32)
           + l1 * (lane == 1).astype(f32)
           + l2 * (lane == 2).astype(f32)
           + c1 * (lane == 3).astype(f32)
           + c2 * (lane == 4).astype(f32))
    part_ref[0] = row


def _hardneg_body(cn1_ref, cn2_ref, part_ref, out1_ref, out2_ref):
    f32 = jnp.float32
    parts = part_ref[...]
    npos = parts[:, 0:1]
    l1_tot = jnp.sum(parts[:, 1:2])
    l2_tot = jnp.sum(parts[:, 2:3])
    c1_tot = jnp.sum(parts[:, 3:4])
    c2_tot = jnp.sum(parts[:, 4:5])
    np_tot = jnp.sum(npos)
    m = npos * float(_NEG_POS_RATIO)

    def topm_sum(v):
        t = jnp.zeros((_B, 1), jnp.int32)
        for bit in range(30, -1, -1):
            cand = t | (1 << bit)
            tf = lax.bitcast_convert_type(cand, f32)
            cnt = jnp.sum((v >= tf).astype(f32), axis=1, keepdims=True)
            t = jnp.where(cnt >= m, cand, t)
        tf = lax.bitcast_convert_type(t, f32)
        gtf = (v > tf).astype(f32)
        cnt_gt = jnp.sum(gtf, axis=1, keepdims=True)
        hard = jnp.sum(v * gtf, axis=1, keepdims=True) + (m - cnt_gt) * tf
        return jnp.sum(hard)

    h1 = topm_sum(cn1_ref[...])
    h2 = topm_sum(cn2_ref[...])
    o1 = (h1 + c1_tot) / np_tot + _ALPHA * l1_tot / (np_tot * 4.0)
    o2 = (h2 + c2_tot) / np_tot + _ALPHA * l2_tot / (np_tot * 4.0)
    out1_ref[...] = o1.reshape(1, 1)
    out2_ref[...] = o2.reshape(1, 1)


def kernel(predicted_locs1, predicted_scores1, predicted_locs2,
           predicted_scores2, boxes, labels, priors_cxcy):
    B, P, C, K = _B, _P, _C, _K
    priors_t = priors_cxcy.T
    locs1_t = jnp.transpose(predicted_locs1, (0, 2, 1))
    locs2_t = jnp.transpose(predicted_locs2, (0, 2, 1))
    scores1_t = jnp.transpose(predicted_scores1, (0, 2, 1))
    scores2_t = jnp.transpose(predicted_scores2, (0, 2, 1))

    pcx, pcy, pw, ph = (priors_cxcy[:, 0], priors_cxcy[:, 1],
                        priors_cxcy[:, 2], priors_cxcy[:, 3])
    pad = _PP - P
    px0 = jnp.pad(pcx - pw * 0.5, (0, pad))
    py0 = jnp.pad(pcy - ph * 0.5, (0, pad))
    px1 = jnp.pad(pcx + pw * 0.5, (0, pad))
    py1 = jnp.pad(pcy + ph * 0.5, (0, pad))
    areap = (px1 - px0) * (py1 - py0)
    boxes_flat = jnp.pad(boxes, ((0, 0), (0, 16 - K), (0, 0))).reshape(B * 64)
    labels_flat = jnp.pad(labels.astype(jnp.int32),
                          ((0, 0), (0, 16 - K))).reshape(B * 16)
    label_pp, obj_pp = _make_sc_match()(
        px0, py0, px1, py1, areap, boxes_flat, labels_flat)
    label_bp = label_pp[:, :P].reshape(B, 1, P)
    obj_bp = obj_pp[:, :P].reshape(B, 1, P)

    cn1, cn2, part = pl.pallas_call(
        _ce_body,
        grid=(B,),
        in_specs=[
            pl.BlockSpec((4, P), lambda i: (0, 0)),
            pl.BlockSpec((1, K, 4), lambda i: (i, 0, 0)),
            pl.BlockSpec((1, 1, P), lambda i: (i, 0, 0)),
            pl.BlockSpec((1, 1, P), lambda i: (i, 0, 0)),
            pl.BlockSpec((1, 4, P), lambda i: (i, 0, 0)),
            pl.BlockSpec((1, C, P), lambda i: (i, 0, 0)),
            pl.BlockSpec((1, 4, P), lambda i: (i, 0, 0)),
            pl.BlockSpec((1, C, P), lambda i: (i, 0, 0)),
        ],
        out_specs=[
            pl.BlockSpec((1, 1, P), lambda i: (i, 0, 0)),
            pl.BlockSpec((1, 1, P), lambda i: (i, 0, 0)),
            pl.BlockSpec((1, 1, 128), lambda i: (i, 0, 0)),
        ],
        out_shape=[
            jax.ShapeDtypeStruct((B, 1, P), jnp.float32),
            jax.ShapeDtypeStruct((B, 1, P), jnp.float32),
            jax.ShapeDtypeStruct((B, 1, 128), jnp.float32),
        ],
    )(priors_t, boxes, label_bp, obj_bp, locs1_t, scores1_t, locs2_t, scores2_t)

    o1, o2 = pl.pallas_call(
        _hardneg_body,
        in_specs=[
            pl.BlockSpec((B, P), lambda: (0, 0)),
            pl.BlockSpec((B, P), lambda: (0, 0)),
            pl.BlockSpec((B, 128), lambda: (0, 0)),
        ],
        out_specs=[
            pl.BlockSpec((1, 1), lambda: (0, 0)),
            pl.BlockSpec((1, 1), lambda: (0, 0)),
        ],
        out_shape=[
            jax.ShapeDtypeStruct((1, 1), jnp.float32),
            jax.ShapeDtypeStruct((1, 1), jnp.float32),
        ],
    )(cn1.reshape(B, P), cn2.reshape(B, P), part.reshape(B, 128))

    return (o1.reshape(()), o2.reshape(()))


# speedup vs baseline: 1.3595x; 1.3595x over previous
import functools

import jax
import jax.numpy as jnp
from jax import lax
from jax.experimental import pallas as pl
from jax.experimental.pallas import tpu as pltpu
from jax.experimental.pallas import tpu_sc as plsc

_PP = 8736
_NCHUNK = _PP // 16

_B, _P, _C, _K = 32, 8732, 21, 12
_THRESHOLD = 0.5
_NEG_POS_RATIO = 3
_ALPHA = 1.0


def _sc_match_body(px0_h, py0_h, px1_h, py1_h, areap_h, boxes_h, labels_h,
                   label_out, obj_out,
                   px0_v, py0_v, px1_v, py1_v, areap_v,
                   box_v, lab_v, bo_v, bk_v, lo_v):
    i32 = jnp.int32
    f32 = jnp.float32
    wid = lax.axis_index("s") * 2 + lax.axis_index("c")
    pltpu.sync_copy(px0_h, px0_v)
    pltpu.sync_copy(py0_h, py0_v)
    pltpu.sync_copy(px1_h, px1_v)
    pltpu.sync_copy(py1_h, py1_v)
    pltpu.sync_copy(areap_h, areap_v)
    pltpu.sync_copy(boxes_h.at[pl.ds(wid * 64, 64)], box_v)
    pltpu.sync_copy(labels_h.at[pl.ds(wid * 16, 16)], lab_v)

    lane = lax.iota(i32, 16)
    pfeo = []
    f32z = jnp.zeros((16,), jnp.float32)
    i32z = jnp.zeros((16,), i32)
    for g in range(3):
        ks = [4 * g + j for j in range(4)]
        sp = []
        for k in ks:
            bv = box_v[pl.ds((k // 4) * 16, 16)]
            j = (k % 4) * 4
            bx0 = lax.broadcast(bv[j + 0], (16,))
            by0 = lax.broadcast(bv[j + 1], (16,))
            bx1 = lax.broadcast(bv[j + 2], (16,))
            by1 = lax.broadcast(bv[j + 3], (16,))
            sp.append((bx0, by0, bx1, by1, (bx1 - bx0) * (by1 - by0)))

        def body(i, carry, sp=sp, ks=ks, g=g):
            mkvs = list(carry[0])
            mkis = list(carry[1])
            sl = pl.ds(i * 16, 16)
            px0 = px0_v[sl]
            py0 = py0_v[sl]
            px1 = px1_v[sl]
            py1 = py1_v[sl]
            areap = areap_v[sl]
            cur = bo_v[sl]
            curk = bk_v[sl]
            for j, k in enumerate(ks):
                bx0, by0, bx1, by1, areab = sp[j]
                iw = jnp.maximum(jnp.minimum(bx1, px1) - jnp.maximum(bx0, px0), 0.0)
                ih = jnp.maximum(jnp.minimum(by1, py1) - jnp.maximum(by0, py0), 0.0)
                inter = iw * ih
                ov = inter / (areab + areap - inter)
                if g == 0 and j == 0:
                    cur = ov
                    curk = i32z
                else:
                    upd = ov > cur
                    cur = jnp.where(upd, ov, cur)
                    curk = jnp.where(upd, k, curk)
                upd2 = ov > mkvs[j]
                mkvs[j] = jnp.where(upd2, ov, mkvs[j])
                mkis[j] = jnp.where(upd2, i, mkis[j])
            bo_v[sl] = cur
            bk_v[sl] = curk
            return tuple(mkvs), tuple(mkis)

        mkvs, mkis = lax.fori_loop(
            0, _NCHUNK, body,
            ((jnp.full((16,), -1.0, f32),) * 4, (i32z,) * 4))
        for j in range(4):
            mkv = mkvs[j]
            flat = mkis[j] * 16 + lane
            m = mkv[0]
            fi = flat[0]
            for t in range(1, 16):
                vt = mkv[t]
                ft = flat[t]
                take = (vt > m) | ((vt == m) & (ft < fi))
                m = jnp.where(take, vt, m)
                fi = jnp.where(take, ft, fi)
            pfeo.append(fi)

    pfeo_b = [lax.broadcast(p, (16,)) for p in pfeo]
    lv = lab_v[...]
    lab_b = [lax.broadcast(lv[k], (16,)) for k in range(_K)]

    def body3(i, _):
        sl = pl.ds(i * 16, 16)
        bk = bk_v[sl]
        bo = bo_v[sl]
        flat = lax.broadcast(i * 16, (16,)) + lane
        for k in range(_K):
            hit = flat == pfeo_b[k]
            bk = jnp.where(hit, k, bk)
            bo = jnp.where(hit, 1.0, bo)
        lab = jnp.zeros((16,), i32)
        for k in range(_K):
            lab = jnp.where(bk == k, lab_b[k], lab)
        lo_v[sl] = jnp.where(bo < _THRESHOLD, 0, lab)
        bk_v[sl] = bk
        return 0

    lax.fori_loop(0, _NCHUNK, body3, 0)
    pltpu.sync_copy(lo_v, label_out.at[wid])
    pltpu.sync_copy(bk_v, obj_out.at[wid])


def _make_sc_match():
    mesh = plsc.VectorSubcoreMesh(core_axis_name="c", subcore_axis_name="s")
    f32 = jnp.float32
    i32 = jnp.int32
    return functools.partial(
        pl.kernel,
        out_type=[jax.ShapeDtypeStruct((_B, _PP), i32),
                  jax.ShapeDtypeStruct((_B, _PP), i32)],
        mesh=mesh,
        scratch_types=[
            pltpu.VMEM((_PP,), f32), pltpu.VMEM((_PP,), f32),
            pltpu.VMEM((_PP,), f32), pltpu.VMEM((_PP,), f32),
            pltpu.VMEM((_PP,), f32),
            pltpu.VMEM((64,), f32), pltpu.VMEM((16,), i32),
            pltpu.VMEM((_PP,), f32), pltpu.VMEM((_PP,), i32),
            pltpu.VMEM((_PP,), i32),
        ],
    )(_sc_match_body)


def _ce_body(priors_ref, boxes_ref, label_ref, obj_ref,
             locs1_ref, scores1_ref, locs2_ref, scores2_ref,
             cn1_ref, cn2_ref, part_ref):
    f32 = jnp.float32
    K, P, C = _K, _P, _C

    pcx = priors_ref[0:1, :]
    pcy = priors_ref[1:2, :]
    pw = priors_ref[2:3, :]
    ph = priors_ref[3:4, :]

    b = boxes_ref[0]
    bx0 = b[:, 0:1]
    by0 = b[:, 1:2]
    bx1 = b[:, 2:3]
    by1 = b[:, 3:4]

    label = label_ref[0]
    obj = obj_ref[0]
    pos = label > 0
    posf = pos.astype(f32)
    npos = jnp.sum(posf)

    kiota = lax.broadcasted_iota(jnp.int32, (K, P), 0)
    onehotf = (obj == kiota).astype(f32)
    gx0 = jnp.sum(onehotf * bx0, axis=0, keepdims=True)
    gy0 = jnp.sum(onehotf * by0, axis=0, keepdims=True)
    gx1 = jnp.sum(onehotf * bx1, axis=0, keepdims=True)
    gy1 = jnp.sum(onehotf * by1, axis=0, keepdims=True)
    cx = (gx0 + gx1) * 0.5
    cy = (gy0 + gy1) * 0.5
    w = gx1 - gx0
    h = gy1 - gy0
    tl0 = (cx - pcx) / (pw / 10.0)
    tl1 = (cy - pcy) / (ph / 10.0)
    tl2 = jnp.log(w / pw) * 5.0
    tl3 = jnp.log(h / ph) * 5.0

    ciota = lax.broadcasted_iota(jnp.int32, (C, P), 0)

    def branch(locs_ref, scores_ref, cn_ref):
        loc_abs = (jnp.abs(locs_ref[0, 0:1, :] - tl0)
                   + jnp.abs(locs_ref[0, 1:2, :] - tl1)
                   + jnp.abs(locs_ref[0, 2:3, :] - tl2)
                   + jnp.abs(locs_ref[0, 3:4, :] - tl3))
        loc_sum = jnp.sum(loc_abs * posf)
        s = scores_ref[0]
        mx = jnp.max(s, axis=0, keepdims=True)
        lse = jnp.log(jnp.sum(jnp.exp(s - mx), axis=0, keepdims=True)) + mx
        strue = jnp.sum(jnp.where(ciota == label, s, 0.0), axis=0, keepdims=True)
        ce = lse - strue
        cep = jnp.sum(ce * posf)
        cn = jnp.maximum(jnp.where(pos, 0.0, ce), 0.0)
        cn_ref[0] = cn
        return loc_sum, cep

    l1, c1 = branch(locs1_ref, scores1_ref, cn1_ref)
    l2, c2 = branch(locs2_ref, scores2_ref, cn2_ref)

    lane = lax.broadcasted_iota(jnp.int32, (1, 128), 1)
    row = (npos * (lane == 0).astype(f32)
           + l1 * (lane == 1).astype(f32)
           + l2 * (lane == 2).astype(f32)
           + c1 * (lane == 3).astype(f32)
           + c2 * (lane == 4).astype(f32))
    part_ref[0] = row


def _hardneg_body(cn1_ref, cn2_ref, part_ref, out1_ref, out2_ref):
    f32 = jnp.float32
    parts = part_ref[...]
    npos = parts[:, 0:1]
    l1_tot = jnp.sum(parts[:, 1:2])
    l2_tot = jnp.sum(parts[:, 2:3])
    c1_tot = jnp.sum(parts[:, 3:4])
    c2_tot = jnp.sum(parts[:, 4:5])
    np_tot = jnp.sum(npos)
    m = npos * float(_NEG_POS_RATIO)

    def topm_sum(v):
        t = jnp.zeros((_B, 1), jnp.int32)
        for bit in range(30, -1, -1):
            cand = t | (1 << bit)
            tf = lax.bitcast_convert_type(cand, f32)
            cnt = jnp.sum((v >= tf).astype(f32), axis=1, keepdims=True)
            t = jnp.where(cnt >= m, cand, t)
        tf = lax.bitcast_convert_type(t, f32)
        gtf = (v > tf).astype(f32)
        cnt_gt = jnp.sum(gtf, axis=1, keepdims=True)
        hard = jnp.sum(v * gtf, axis=1, keepdims=True) + (m - cnt_gt) * tf
        return jnp.sum(hard)

    h1 = topm_sum(cn1_ref[...])
    h2 = topm_sum(cn2_ref[...])
    o1 = (h1 + c1_tot) / np_tot + _ALPHA * l1_tot / (np_tot * 4.0)
    o2 = (h2 + c2_tot) / np_tot + _ALPHA * l2_tot / (np_tot * 4.0)
    out1_ref[...] = o1.reshape(1, 1)
    out2_ref[...] = o2.reshape(1, 1)


def kernel(predicted_locs1, predicted_scores1, predicted_locs2,
           predicted_scores2, boxes, labels, priors_cxcy):
    B, P, C, K = _B, _P, _C, _K
    priors_t = priors_cxcy.T
    locs1_t = jnp.transpose(predicted_locs1, (0, 2, 1))
    locs2_t = jnp.transpose(predicted_locs2, (0, 2, 1))
    scores1_t = jnp.transpose(predicted_scores1, (0, 2, 1))
    scores2_t = jnp.transpose(predicted_scores2, (0, 2, 1))

    pcx, pcy, pw, ph = (priors_cxcy[:, 0], priors_cxcy[:, 1],
                        priors_cxcy[:, 2], priors_cxcy[:, 3])
    pad = _PP - P
    px0 = jnp.pad(pcx - pw * 0.5, (0, pad))
    py0 = jnp.pad(pcy - ph * 0.5, (0, pad))
    px1 = jnp.pad(pcx + pw * 0.5, (0, pad))
    py1 = jnp.pad(pcy + ph * 0.5, (0, pad))
    areap = (px1 - px0) * (py1 - py0)
    boxes_flat = jnp.pad(boxes, ((0, 0), (0, 16 - K), (0, 0))).reshape(B * 64)
    labels_flat = jnp.pad(labels.astype(jnp.int32),
                          ((0, 0), (0, 16 - K))).reshape(B * 16)
    label_pp, obj_pp = _make_sc_match()(
        px0, py0, px1, py1, areap, boxes_flat, labels_flat)
    label_bp = label_pp[:, :P].reshape(B, 1, P)
    obj_bp = obj_pp[:, :P].reshape(B, 1, P)

    cn1, cn2, part = pl.pallas_call(
        _ce_body,
        grid=(B,),
        in_specs=[
            pl.BlockSpec((4, P), lambda i: (0, 0)),
            pl.BlockSpec((1, K, 4), lambda i: (i, 0, 0)),
            pl.BlockSpec((1, 1, P), lambda i: (i, 0, 0)),
            pl.BlockSpec((1, 1, P), lambda i: (i, 0, 0)),
            pl.BlockSpec((1, 4, P), lambda i: (i, 0, 0)),
            pl.BlockSpec((1, C, P), lambda i: (i, 0, 0)),
            pl.BlockSpec((1, 4, P), lambda i: (i, 0, 0)),
            pl.BlockSpec((1, C, P), lambda i: (i, 0, 0)),
        ],
        out_specs=[
            pl.BlockSpec((1, 1, P), lambda i: (i, 0, 0)),
            pl.BlockSpec((1, 1, P), lambda i: (i, 0, 0)),
            pl.BlockSpec((1, 1, 128), lambda i: (i, 0, 0)),
        ],
        out_shape=[
            jax.ShapeDtypeStruct((B, 1, P), jnp.float32),
            jax.ShapeDtypeStruct((B, 1, P), jnp.float32),
            jax.ShapeDtypeStruct((B, 1, 128), jnp.float32),
        ],
    )(priors_t, boxes, label_bp, obj_bp, locs1_t, scores1_t, locs2_t, scores2_t)

    o1, o2 = pl.pallas_call(
        _hardneg_body,
        in_specs=[
            pl.BlockSpec((B, P), lambda: (0, 0)),
            pl.BlockSpec((B, P), lambda: (0, 0)),
            pl.BlockSpec((B, 128), lambda: (0, 0)),
        ],
        out_specs=[
            pl.BlockSpec((1, 1), lambda: (0, 0)),
            pl.BlockSpec((1, 1), lambda: (0, 0)),
        ],
        out_shape=[
            jax.ShapeDtypeStruct((1, 1), jnp.float32),
            jax.ShapeDtypeStruct((1, 1), jnp.float32),
        ],
    )(cn1.reshape(B, P), cn2.reshape(B, P), part.reshape(B, 128))

    return (o1.reshape(()), o2.reshape(()))


# speedup vs baseline: 1.4340x; 1.0548x over previous
import functools

import jax
import jax.numpy as jnp
from jax import lax
from jax.experimental import pallas as pl
from jax.experimental.pallas import tpu as pltpu
from jax.experimental.pallas import tpu_sc as plsc

_PP = 8736
_NCHUNK = _PP // 16

_B, _P, _C, _K = 32, 8732, 21, 12
_THRESHOLD = 0.5
_NEG_POS_RATIO = 3
_ALPHA = 1.0


def _sc_match_body(px0_h, py0_h, px1_h, py1_h, areap_h, boxes_h, labels_h,
                   label_out, obj_out,
                   px0_v, py0_v, px1_v, py1_v, areap_v,
                   box_v, lab_v, bo_v, bk_v, lo_v):
    i32 = jnp.int32
    f32 = jnp.float32
    wid = lax.axis_index("s") * 2 + lax.axis_index("c")
    pltpu.sync_copy(px0_h, px0_v)
    pltpu.sync_copy(py0_h, py0_v)
    pltpu.sync_copy(px1_h, px1_v)
    pltpu.sync_copy(py1_h, py1_v)
    pltpu.sync_copy(areap_h, areap_v)
    pltpu.sync_copy(boxes_h.at[pl.ds(wid * 64, 64)], box_v)
    pltpu.sync_copy(labels_h.at[pl.ds(wid * 16, 16)], lab_v)

    lane = lax.iota(i32, 16)
    pfeo = []
    f32z = jnp.zeros((16,), jnp.float32)
    i32z = jnp.zeros((16,), i32)
    for g in range(3):
        ks = [4 * g + j for j in range(4)]
        sp = []
        for k in ks:
            bv = box_v[pl.ds((k // 4) * 16, 16)]
            j = (k % 4) * 4
            bx0 = lax.broadcast(bv[j + 0], (16,))
            by0 = lax.broadcast(bv[j + 1], (16,))
            bx1 = lax.broadcast(bv[j + 2], (16,))
            by1 = lax.broadcast(bv[j + 3], (16,))
            sp.append((bx0, by0, bx1, by1, (bx1 - bx0) * (by1 - by0)))

        def body(i2, carry, sp=sp, ks=ks, g=g):
            mkvs = list(carry[0])
            mkis = list(carry[1])
            for u in range(2):
                i = i2 * 2 + u
                sl = pl.ds(i * 16, 16)
                px0 = px0_v[sl]
                py0 = py0_v[sl]
                px1 = px1_v[sl]
                py1 = py1_v[sl]
                areap = areap_v[sl]
                cur = bo_v[sl]
                curk = bk_v[sl]
                for j, k in enumerate(ks):
                    bx0, by0, bx1, by1, areab = sp[j]
                    iw = jnp.maximum(jnp.minimum(bx1, px1) - jnp.maximum(bx0, px0), 0.0)
                    ih = jnp.maximum(jnp.minimum(by1, py1) - jnp.maximum(by0, py0), 0.0)
                    inter = iw * ih
                    ov = inter / (areab + areap - inter)
                    if g == 0 and j == 0:
                        cur = ov
                        curk = i32z
                    else:
                        upd = ov > cur
                        cur = jnp.where(upd, ov, cur)
                        curk = jnp.where(upd, k, curk)
                    upd2 = ov > mkvs[j]
                    mkvs[j] = jnp.where(upd2, ov, mkvs[j])
                    mkis[j] = jnp.where(upd2, i, mkis[j])
                bo_v[sl] = cur
                bk_v[sl] = curk
            return tuple(mkvs), tuple(mkis)

        mkvs, mkis = lax.fori_loop(
            0, _NCHUNK // 2, body,
            ((jnp.full((16,), -1.0, f32),) * 4, (i32z,) * 4))
        for j in range(4):
            mkv = mkvs[j]
            flat = mkis[j] * 16 + lane
            m = mkv[0]
            fi = flat[0]
            for t in range(1, 16):
                vt = mkv[t]
                ft = flat[t]
                take = (vt > m) | ((vt == m) & (ft < fi))
                m = jnp.where(take, vt, m)
                fi = jnp.where(take, ft, fi)
            pfeo.append(fi)

    pfeo_b = [lax.broadcast(p, (16,)) for p in pfeo]
    lv = lab_v[...]
    lab_b = [lax.broadcast(lv[k], (16,)) for k in range(_K)]

    def body3(i, _):
        sl = pl.ds(i * 16, 16)
        bk = bk_v[sl]
        bo = bo_v[sl]
        flat = lax.broadcast(i * 16, (16,)) + lane
        for k in range(_K):
            hit = flat == pfeo_b[k]
            bk = jnp.where(hit, k, bk)
            bo = jnp.where(hit, 1.0, bo)
        lab = jnp.zeros((16,), i32)
        for k in range(_K):
            lab = jnp.where(bk == k, lab_b[k], lab)
        lo_v[sl] = jnp.where(bo < _THRESHOLD, 0, lab)
        bk_v[sl] = bk
        return 0

    lax.fori_loop(0, _NCHUNK, body3, 0)
    pltpu.sync_copy(lo_v, label_out.at[wid])
    pltpu.sync_copy(bk_v, obj_out.at[wid])


def _make_sc_match():
    mesh = plsc.VectorSubcoreMesh(core_axis_name="c", subcore_axis_name="s")
    f32 = jnp.float32
    i32 = jnp.int32
    return functools.partial(
        pl.kernel,
        out_type=[jax.ShapeDtypeStruct((_B, _PP), i32),
                  jax.ShapeDtypeStruct((_B, _PP), i32)],
        mesh=mesh,
        scratch_types=[
            pltpu.VMEM((_PP,), f32), pltpu.VMEM((_PP,), f32),
            pltpu.VMEM((_PP,), f32), pltpu.VMEM((_PP,), f32),
            pltpu.VMEM((_PP,), f32),
            pltpu.VMEM((64,), f32), pltpu.VMEM((16,), i32),
            pltpu.VMEM((_PP,), f32), pltpu.VMEM((_PP,), i32),
            pltpu.VMEM((_PP,), i32),
        ],
    )(_sc_match_body)


def _ce_body(priors_ref, boxes_ref, label_ref, obj_ref,
             locs1_ref, scores1_ref, locs2_ref, scores2_ref,
             cn1_ref, cn2_ref, part_ref):
    f32 = jnp.float32
    K, P, C = _K, _P, _C

    pcx = priors_ref[0:1, :]
    pcy = priors_ref[1:2, :]
    pw = priors_ref[2:3, :]
    ph = priors_ref[3:4, :]

    b = boxes_ref[0]
    bx0 = b[:, 0:1]
    by0 = b[:, 1:2]
    bx1 = b[:, 2:3]
    by1 = b[:, 3:4]

    label = label_ref[0]
    obj = obj_ref[0]
    pos = label > 0
    posf = pos.astype(f32)
    npos = jnp.sum(posf)

    kiota = lax.broadcasted_iota(jnp.int32, (K, P), 0)
    onehotf = (obj == kiota).astype(f32)
    gx0 = jnp.sum(onehotf * bx0, axis=0, keepdims=True)
    gy0 = jnp.sum(onehotf * by0, axis=0, keepdims=True)
    gx1 = jnp.sum(onehotf * bx1, axis=0, keepdims=True)
    gy1 = jnp.sum(onehotf * by1, axis=0, keepdims=True)
    cx = (gx0 + gx1) * 0.5
    cy = (gy0 + gy1) * 0.5
    w = gx1 - gx0
    h = gy1 - gy0
    tl0 = (cx - pcx) / (pw / 10.0)
    tl1 = (cy - pcy) / (ph / 10.0)
    tl2 = jnp.log(w / pw) * 5.0
    tl3 = jnp.log(h / ph) * 5.0

    ciota = lax.broadcasted_iota(jnp.int32, (C, P), 0)

    def branch(locs_ref, scores_ref, cn_ref):
        loc_abs = (jnp.abs(locs_ref[0, 0:1, :] - tl0)
                   + jnp.abs(locs_ref[0, 1:2, :] - tl1)
                   + jnp.abs(locs_ref[0, 2:3, :] - tl2)
                   + jnp.abs(locs_ref[0, 3:4, :] - tl3))
        loc_sum = jnp.sum(loc_abs * posf)
        s = scores_ref[0]
        mx = jnp.max(s, axis=0, keepdims=True)
        lse = jnp.log(jnp.sum(jnp.exp(s - mx), axis=0, keepdims=True)) + mx
        strue = jnp.sum(jnp.where(ciota == label, s, 0.0), axis=0, keepdims=True)
        ce = lse - strue
        cep = jnp.sum(ce * posf)
        cn = jnp.maximum(jnp.where(pos, 0.0, ce), 0.0)
        cn_ref[0] = cn
        return loc_sum, cep

    l1, c1 = branch(locs1_ref, scores1_ref, cn1_ref)
    l2, c2 = branch(locs2_ref, scores2_ref, cn2_ref)

    lane = lax.broadcasted_iota(jnp.int32, (1, 128), 1)
    row = (npos * (lane == 0).astype(f32)
           + l1 * (lane == 1).astype(f32)
           + l2 * (lane == 2).astype(f32)
           + c1 * (lane == 3).astype(f32)
           + c2 * (lane == 4).astype(f32))
    part_ref[0] = row


def _hardneg_body(cn1_ref, cn2_ref, part_ref, out1_ref, out2_ref):
    f32 = jnp.float32
    parts = part_ref[...]
    npos = parts[:, 0:1]
    l1_tot = jnp.sum(parts[:, 1:2])
    l2_tot = jnp.sum(parts[:, 2:3])
    c1_tot = jnp.sum(parts[:, 3:4])
    c2_tot = jnp.sum(parts[:, 4:5])
    np_tot = jnp.sum(npos)
    m = npos * float(_NEG_POS_RATIO)

    def topm_sum(v):
        t = jnp.zeros((_B, 1), jnp.int32)
        for bit in range(30, -1, -1):
            cand = t | (1 << bit)
            tf = lax.bitcast_convert_type(cand, f32)
            cnt = jnp.sum((v >= tf).astype(f32), axis=1, keepdims=True)
            t = jnp.where(cnt >= m, cand, t)
        tf = lax.bitcast_convert_type(t, f32)
        gtf = (v > tf).astype(f32)
        cnt_gt = jnp.sum(gtf, axis=1, keepdims=True)
        hard = jnp.sum(v * gtf, axis=1, keepdims=True) + (m - cnt_gt) * tf
        return jnp.sum(hard)

    h1 = topm_sum(cn1_ref[...])
    h2 = topm_sum(cn2_ref[...])
    o1 = (h1 + c1_tot) / np_tot + _ALPHA * l1_tot / (np_tot * 4.0)
    o2 = (h2 + c2_tot) / np_tot + _ALPHA * l2_tot / (np_tot * 4.0)
    out1_ref[...] = o1.reshape(1, 1)
    out2_ref[...] = o2.reshape(1, 1)


def kernel(predicted_locs1, predicted_scores1, predicted_locs2,
           predicted_scores2, boxes, labels, priors_cxcy):
    B, P, C, K = _B, _P, _C, _K
    priors_t = priors_cxcy.T
    locs1_t = jnp.transpose(predicted_locs1, (0, 2, 1))
    locs2_t = jnp.transpose(predicted_locs2, (0, 2, 1))
    scores1_t = jnp.transpose(predicted_scores1, (0, 2, 1))
    scores2_t = jnp.transpose(predicted_scores2, (0, 2, 1))

    pcx, pcy, pw, ph = (priors_cxcy[:, 0], priors_cxcy[:, 1],
                        priors_cxcy[:, 2], priors_cxcy[:, 3])
    pad = _PP - P
    px0 = jnp.pad(pcx - pw * 0.5, (0, pad))
    py0 = jnp.pad(pcy - ph * 0.5, (0, pad))
    px1 = jnp.pad(pcx + pw * 0.5, (0, pad))
    py1 = jnp.pad(pcy + ph * 0.5, (0, pad))
    areap = (px1 - px0) * (py1 - py0)
    boxes_flat = jnp.pad(boxes, ((0, 0), (0, 16 - K), (0, 0))).reshape(B * 64)
    labels_flat = jnp.pad(labels.astype(jnp.int32),
                          ((0, 0), (0, 16 - K))).reshape(B * 16)
    label_pp, obj_pp = _make_sc_match()(
        px0, py0, px1, py1, areap, boxes_flat, labels_flat)
    label_bp = label_pp[:, :P].reshape(B, 1, P)
    obj_bp = obj_pp[:, :P].reshape(B, 1, P)

    cn1, cn2, part = pl.pallas_call(
        _ce_body,
        grid=(B,),
        in_specs=[
            pl.BlockSpec((4, P), lambda i: (0, 0)),
            pl.BlockSpec((1, K, 4), lambda i: (i, 0, 0)),
            pl.BlockSpec((1, 1, P), lambda i: (i, 0, 0)),
            pl.BlockSpec((1, 1, P), lambda i: (i, 0, 0)),
            pl.BlockSpec((1, 4, P), lambda i: (i, 0, 0)),
            pl.BlockSpec((1, C, P), lambda i: (i, 0, 0)),
            pl.BlockSpec((1, 4, P), lambda i: (i, 0, 0)),
            pl.BlockSpec((1, C, P), lambda i: (i, 0, 0)),
        ],
        out_specs=[
            pl.BlockSpec((1, 1, P), lambda i: (i, 0, 0)),
            pl.BlockSpec((1, 1, P), lambda i: (i, 0, 0)),
            pl.BlockSpec((1, 1, 128), lambda i: (i, 0, 0)),
        ],
        out_shape=[
            jax.ShapeDtypeStruct((B, 1, P), jnp.float32),
            jax.ShapeDtypeStruct((B, 1, P), jnp.float32),
            jax.ShapeDtypeStruct((B, 1, 128), jnp.float32),
        ],
    )(priors_t, boxes, label_bp, obj_bp, locs1_t, scores1_t, locs2_t, scores2_t)

    o1, o2 = pl.pallas_call(
        _hardneg_body,
        in_specs=[
            pl.BlockSpec((B, P), lambda: (0, 0)),
            pl.BlockSpec((B, P), lambda: (0, 0)),
            pl.BlockSpec((B, 128), lambda: (0, 0)),
        ],
        out_specs=[
            pl.BlockSpec((1, 1), lambda: (0, 0)),
            pl.BlockSpec((1, 1), lambda: (0, 0)),
        ],
        out_shape=[
            jax.ShapeDtypeStruct((1, 1), jnp.float32),
            jax.ShapeDtypeStruct((1, 1), jnp.float32),
        ],
    )(cn1.reshape(B, P), cn2.reshape(B, P), part.reshape(B, 128))

    return (o1.reshape(()), o2.reshape(()))


# speedup vs baseline: 1.4366x; 1.0018x over previous
import functools

import jax
import jax.numpy as jnp
from jax import lax
from jax.experimental import pallas as pl
from jax.experimental.pallas import tpu as pltpu
from jax.experimental.pallas import tpu_sc as plsc

_PP = 8736
_NCHUNK = _PP // 16

_B, _P, _C, _K = 32, 8732, 21, 12
_THRESHOLD = 0.5
_NEG_POS_RATIO = 3
_ALPHA = 1.0


def _sc_match_body(px0_h, py0_h, px1_h, py1_h, areap_h, boxes_h, labels_h,
                   label_out, obj_out,
                   px0_v, py0_v, px1_v, py1_v, areap_v,
                   box_v, lab_v, bo_v, bk_v, lo_v):
    i32 = jnp.int32
    f32 = jnp.float32
    wid = lax.axis_index("s") * 2 + lax.axis_index("c")
    pltpu.sync_copy(px0_h, px0_v)
    pltpu.sync_copy(py0_h, py0_v)
    pltpu.sync_copy(px1_h, px1_v)
    pltpu.sync_copy(py1_h, py1_v)
    pltpu.sync_copy(areap_h, areap_v)
    pltpu.sync_copy(boxes_h.at[pl.ds(wid * 64, 64)], box_v)
    pltpu.sync_copy(labels_h.at[pl.ds(wid * 16, 16)], lab_v)

    lane = lax.iota(i32, 16)
    pfeo = []
    f32z = jnp.zeros((16,), jnp.float32)
    i32z = jnp.zeros((16,), i32)
    for g in range(3):
        ks = [4 * g + j for j in range(4)]
        sp = []
        for k in ks:
            bv = box_v[pl.ds((k // 4) * 16, 16)]
            j = (k % 4) * 4
            bx0 = lax.broadcast(bv[j + 0], (16,))
            by0 = lax.broadcast(bv[j + 1], (16,))
            bx1 = lax.broadcast(bv[j + 2], (16,))
            by1 = lax.broadcast(bv[j + 3], (16,))
            sp.append((bx0, by0, bx1, by1, (bx1 - bx0) * (by1 - by0)))

        def body(i2, carry, sp=sp, ks=ks, g=g):
            mkvs = list(carry[0])
            mkis = list(carry[1])
            for u in range(2):
                i = i2 * 2 + u
                sl = pl.ds(i * 16, 16)
                px0 = px0_v[sl]
                py0 = py0_v[sl]
                px1 = px1_v[sl]
                py1 = py1_v[sl]
                areap = areap_v[sl]
                if g == 0:
                    cur = f32z
                    curk = i32z
                else:
                    cur = bo_v[sl]
                    curk = bk_v[sl]
                for j, k in enumerate(ks):
                    bx0, by0, bx1, by1, areab = sp[j]
                    iw = jnp.maximum(jnp.minimum(bx1, px1) - jnp.maximum(bx0, px0), 0.0)
                    ih = jnp.maximum(jnp.minimum(by1, py1) - jnp.maximum(by0, py0), 0.0)
                    inter = iw * ih
                    ov = inter / (areab + areap - inter)
                    if g == 0 and j == 0:
                        cur = ov
                        curk = i32z
                    else:
                        upd = ov > cur
                        cur = jnp.where(upd, ov, cur)
                        curk = jnp.where(upd, k, curk)
                    upd2 = ov > mkvs[j]
                    mkvs[j] = jnp.where(upd2, ov, mkvs[j])
                    mkis[j] = jnp.where(upd2, i, mkis[j])
                bo_v[sl] = cur
                bk_v[sl] = curk
            return tuple(mkvs), tuple(mkis)

        mkvs, mkis = lax.fori_loop(
            0, _NCHUNK // 2, body,
            ((jnp.full((16,), -1.0, f32),) * 4, (i32z,) * 4))
        for j in range(4):
            mkv = mkvs[j]
            flat = mkis[j] * 16 + lane
            m = mkv[0]
            fi = flat[0]
            for t in range(1, 16):
                vt = mkv[t]
                ft = flat[t]
                take = (vt > m) | ((vt == m) & (ft < fi))
                m = jnp.where(take, vt, m)
                fi = jnp.where(take, ft, fi)
            pfeo.append(fi)

    pfeo_b = [lax.broadcast(p, (16,)) for p in pfeo]
    lv = lab_v[...]
    lab_b = [lax.broadcast(lv[k], (16,)) for k in range(_K)]

    def body3(i2, _):
        for u in range(2):
            i = i2 * 2 + u
            sl = pl.ds(i * 16, 16)
            bk = bk_v[sl]
            bo = bo_v[sl]
            flat = lax.broadcast(i * 16, (16,)) + lane
            for k in range(_K):
                hit = flat == pfeo_b[k]
                bk = jnp.where(hit, k, bk)
                bo = jnp.where(hit, 1.0, bo)
            lab = jnp.zeros((16,), i32)
            for k in range(_K):
                lab = jnp.where(bk == k, lab_b[k], lab)
            lo_v[sl] = jnp.where(bo < _THRESHOLD, 0, lab)
            bk_v[sl] = bk
        return 0

    lax.fori_loop(0, _NCHUNK // 2, body3, 0)
    pltpu.sync_copy(lo_v, label_out.at[wid])
    pltpu.sync_copy(bk_v, obj_out.at[wid])


def _make_sc_match():
    mesh = plsc.VectorSubcoreMesh(core_axis_name="c", subcore_axis_name="s")
    f32 = jnp.float32
    i32 = jnp.int32
    return functools.partial(
        pl.kernel,
        out_type=[jax.ShapeDtypeStruct((_B, _PP), i32),
                  jax.ShapeDtypeStruct((_B, _PP), i32)],
        mesh=mesh,
        scratch_types=[
            pltpu.VMEM((_PP,), f32), pltpu.VMEM((_PP,), f32),
            pltpu.VMEM((_PP,), f32), pltpu.VMEM((_PP,), f32),
            pltpu.VMEM((_PP,), f32),
            pltpu.VMEM((64,), f32), pltpu.VMEM((16,), i32),
            pltpu.VMEM((_PP,), f32), pltpu.VMEM((_PP,), i32),
            pltpu.VMEM((_PP,), i32),
        ],
    )(_sc_match_body)


def _ce_body(priors_ref, boxes_ref, label_ref, obj_ref,
             locs1_ref, scores1_ref, locs2_ref, scores2_ref,
             cn1_ref, cn2_ref, part_ref):
    f32 = jnp.float32
    K, P, C = _K, _P, _C

    pcx = priors_ref[0:1, :]
    pcy = priors_ref[1:2, :]
    pw = priors_ref[2:3, :]
    ph = priors_ref[3:4, :]

    b = boxes_ref[0]
    bx0 = b[:, 0:1]
    by0 = b[:, 1:2]
    bx1 = b[:, 2:3]
    by1 = b[:, 3:4]

    label = label_ref[0]
    obj = obj_ref[0]
    pos = label > 0
    posf = pos.astype(f32)
    npos = jnp.sum(posf)

    kiota = lax.broadcasted_iota(jnp.int32, (K, P), 0)
    onehotf = (obj == kiota).astype(f32)
    gx0 = jnp.sum(onehotf * bx0, axis=0, keepdims=True)
    gy0 = jnp.sum(onehotf * by0, axis=0, keepdims=True)
    gx1 = jnp.sum(onehotf * bx1, axis=0, keepdims=True)
    gy1 = jnp.sum(onehotf * by1, axis=0, keepdims=True)
    cx = (gx0 + gx1) * 0.5
    cy = (gy0 + gy1) * 0.5
    w = gx1 - gx0
    h = gy1 - gy0
    tl0 = (cx - pcx) / (pw / 10.0)
    tl1 = (cy - pcy) / (ph / 10.0)
    tl2 = jnp.log(w / pw) * 5.0
    tl3 = jnp.log(h / ph) * 5.0

    ciota = lax.broadcasted_iota(jnp.int32, (C, P), 0)

    def branch(locs_ref, scores_ref, cn_ref):
        loc_abs = (jnp.abs(locs_ref[0, 0:1, :] - tl0)
                   + jnp.abs(locs_ref[0, 1:2, :] - tl1)
                   + jnp.abs(locs_ref[0, 2:3, :] - tl2)
                   + jnp.abs(locs_ref[0, 3:4, :] - tl3))
        loc_sum = jnp.sum(loc_abs * posf)
        s = scores_ref[0]
        mx = jnp.max(s, axis=0, keepdims=True)
        lse = jnp.log(jnp.sum(jnp.exp(s - mx), axis=0, keepdims=True)) + mx
        strue = jnp.sum(jnp.where(ciota == label, s, 0.0), axis=0, keepdims=True)
        ce = lse - strue
        cep = jnp.sum(ce * posf)
        cn = jnp.maximum(jnp.where(pos, 0.0, ce), 0.0)
        cn_ref[0] = cn
        return loc_sum, cep

    l1, c1 = branch(locs1_ref, scores1_ref, cn1_ref)
    l2, c2 = branch(locs2_ref, scores2_ref, cn2_ref)

    lane = lax.broadcasted_iota(jnp.int32, (1, 128), 1)
    row = (npos * (lane == 0).astype(f32)
           + l1 * (lane == 1).astype(f32)
           + l2 * (lane == 2).astype(f32)
           + c1 * (lane == 3).astype(f32)
           + c2 * (lane == 4).astype(f32))
    part_ref[0] = row


def _hardneg_body(cn1_ref, cn2_ref, part_ref, out1_ref, out2_ref):
    f32 = jnp.float32
    parts = part_ref[...]
    npos = parts[:, 0:1]
    l1_tot = jnp.sum(parts[:, 1:2])
    l2_tot = jnp.sum(parts[:, 2:3])
    c1_tot = jnp.sum(parts[:, 3:4])
    c2_tot = jnp.sum(parts[:, 4:5])
    np_tot = jnp.sum(npos)
    m = npos * float(_NEG_POS_RATIO)

    def topm_sum(v):
        t = jnp.zeros((_B, 1), jnp.int32)
        for bit in range(30, -1, -1):
            cand = t | (1 << bit)
            tf = lax.bitcast_convert_type(cand, f32)
            cnt = jnp.sum((v >= tf).astype(f32), axis=1, keepdims=True)
            t = jnp.where(cnt >= m, cand, t)
        tf = lax.bitcast_convert_type(t, f32)
        gtf = (v > tf).astype(f32)
        cnt_gt = jnp.sum(gtf, axis=1, keepdims=True)
        hard = jnp.sum(v * gtf, axis=1, keepdims=True) + (m - cnt_gt) * tf
        return jnp.sum(hard)

    h1 = topm_sum(cn1_ref[...])
    h2 = topm_sum(cn2_ref[...])
    o1 = (h1 + c1_tot) / np_tot + _ALPHA * l1_tot / (np_tot * 4.0)
    o2 = (h2 + c2_tot) / np_tot + _ALPHA * l2_tot / (np_tot * 4.0)
    out1_ref[...] = o1.reshape(1, 1)
    out2_ref[...] = o2.reshape(1, 1)


def kernel(predicted_locs1, predicted_scores1, predicted_locs2,
           predicted_scores2, boxes, labels, priors_cxcy):
    B, P, C, K = _B, _P, _C, _K
    priors_t = priors_cxcy.T
    locs1_t = jnp.transpose(predicted_locs1, (0, 2, 1))
    locs2_t = jnp.transpose(predicted_locs2, (0, 2, 1))
    scores1_t = jnp.transpose(predicted_scores1, (0, 2, 1))
    scores2_t = jnp.transpose(predicted_scores2, (0, 2, 1))

    pcx, pcy, pw, ph = (priors_cxcy[:, 0], priors_cxcy[:, 1],
                        priors_cxcy[:, 2], priors_cxcy[:, 3])
    pad = _PP - P
    px0 = jnp.pad(pcx - pw * 0.5, (0, pad))
    py0 = jnp.pad(pcy - ph * 0.5, (0, pad))
    px1 = jnp.pad(pcx + pw * 0.5, (0, pad))
    py1 = jnp.pad(pcy + ph * 0.5, (0, pad))
    areap = (px1 - px0) * (py1 - py0)
    boxes_flat = jnp.pad(boxes, ((0, 0), (0, 16 - K), (0, 0))).reshape(B * 64)
    labels_flat = jnp.pad(labels.astype(jnp.int32),
                          ((0, 0), (0, 16 - K))).reshape(B * 16)
    label_pp, obj_pp = _make_sc_match()(
        px0, py0, px1, py1, areap, boxes_flat, labels_flat)
    label_bp = label_pp[:, :P].reshape(B, 1, P)
    obj_bp = obj_pp[:, :P].reshape(B, 1, P)

    cn1, cn2, part = pl.pallas_call(
        _ce_body,
        grid=(B,),
        in_specs=[
            pl.BlockSpec((4, P), lambda i: (0, 0)),
            pl.BlockSpec((1, K, 4), lambda i: (i, 0, 0)),
            pl.BlockSpec((1, 1, P), lambda i: (i, 0, 0)),
            pl.BlockSpec((1, 1, P), lambda i: (i, 0, 0)),
            pl.BlockSpec((1, 4, P), lambda i: (i, 0, 0)),
            pl.BlockSpec((1, C, P), lambda i: (i, 0, 0)),
            pl.BlockSpec((1, 4, P), lambda i: (i, 0, 0)),
            pl.BlockSpec((1, C, P), lambda i: (i, 0, 0)),
        ],
        out_specs=[
            pl.BlockSpec((1, 1, P), lambda i: (i, 0, 0)),
            pl.BlockSpec((1, 1, P), lambda i: (i, 0, 0)),
            pl.BlockSpec((1, 1, 128), lambda i: (i, 0, 0)),
        ],
        out_shape=[
            jax.ShapeDtypeStruct((B, 1, P), jnp.float32),
            jax.ShapeDtypeStruct((B, 1, P), jnp.float32),
            jax.ShapeDtypeStruct((B, 1, 128), jnp.float32),
        ],
    )(priors_t, boxes, label_bp, obj_bp, locs1_t, scores1_t, locs2_t, scores2_t)

    o1, o2 = pl.pallas_call(
        _hardneg_body,
        in_specs=[
            pl.BlockSpec((B, P), lambda: (0, 0)),
            pl.BlockSpec((B, P), lambda: (0, 0)),
            pl.BlockSpec((B, 128), lambda: (0, 0)),
        ],
        out_specs=[
            pl.BlockSpec((1, 1), lambda: (0, 0)),
            pl.BlockSpec((1, 1), lambda: (0, 0)),
        ],
        out_shape=[
            jax.ShapeDtypeStruct((1, 1), jnp.float32),
            jax.ShapeDtypeStruct((1, 1), jnp.float32),
        ],
    )(cn1.reshape(B, P), cn2.reshape(B, P), part.reshape(B, 128))

    return (o1.reshape(()), o2.reshape(()))


# speedup vs baseline: 1.6663x; 1.1599x over previous
import functools

import jax
import jax.numpy as jnp
from jax import lax
from jax.experimental import pallas as pl
from jax.experimental.pallas import tpu as pltpu
from jax.experimental.pallas import tpu_sc as plsc

_PP = 8736
_NCHUNK = _PP // 16

_B, _P, _C, _K = 32, 8732, 21, 12
_THRESHOLD = 0.5
_NEG_POS_RATIO = 3
_ALPHA = 1.0


def _sc_match_body(px0_h, py0_h, px1_h, py1_h, areap_h, boxes_h, labels_h,
                   label_out, obj_out,
                   px0_v, py0_v, px1_v, py1_v, areap_v,
                   box_v, lab_v, bo_v, bk_v, lo_v):
    i32 = jnp.int32
    f32 = jnp.float32
    wid = lax.axis_index("s") * 2 + lax.axis_index("c")
    pltpu.sync_copy(px0_h, px0_v)
    pltpu.sync_copy(py0_h, py0_v)
    pltpu.sync_copy(px1_h, px1_v)
    pltpu.sync_copy(py1_h, py1_v)
    pltpu.sync_copy(areap_h, areap_v)
    pltpu.sync_copy(boxes_h.at[pl.ds(wid * 64, 64)], box_v)
    pltpu.sync_copy(labels_h.at[pl.ds(wid * 16, 16)], lab_v)

    lane = lax.iota(i32, 16)
    pfeo = []
    f32z = jnp.zeros((16,), jnp.float32)
    i32z = jnp.zeros((16,), i32)
    for g in range(3):
        ks = [4 * g + j for j in range(4)]
        sp = []
        for k in ks:
            bv = box_v[pl.ds((k // 4) * 16, 16)]
            j = (k % 4) * 4
            bx0 = lax.broadcast(bv[j + 0], (16,))
            by0 = lax.broadcast(bv[j + 1], (16,))
            bx1 = lax.broadcast(bv[j + 2], (16,))
            by1 = lax.broadcast(bv[j + 3], (16,))
            sp.append((bx0, by0, bx1, by1, (bx1 - bx0) * (by1 - by0)))

        def body(i2, carry, sp=sp, ks=ks, g=g):
            mkvs = list(carry[0])
            mkis = list(carry[1])
            for u in range(2):
                i = i2 * 2 + u
                sl = pl.ds(i * 16, 16)
                px0 = px0_v[sl]
                py0 = py0_v[sl]
                px1 = px1_v[sl]
                py1 = py1_v[sl]
                areap = areap_v[sl]
                if g == 0:
                    cur = f32z
                    curk = i32z
                else:
                    cur = bo_v[sl]
                    curk = bk_v[sl]
                for j, k in enumerate(ks):
                    bx0, by0, bx1, by1, areab = sp[j]
                    iw = jnp.maximum(jnp.minimum(bx1, px1) - jnp.maximum(bx0, px0), 0.0)
                    ih = jnp.maximum(jnp.minimum(by1, py1) - jnp.maximum(by0, py0), 0.0)
                    inter = iw * ih
                    ov = inter / (areab + areap - inter)
                    if g == 0 and j == 0:
                        cur = ov
                        curk = i32z
                    else:
                        upd = ov > cur
                        cur = jnp.where(upd, ov, cur)
                        curk = jnp.where(upd, k, curk)
                    upd2 = ov > mkvs[j]
                    mkvs[j] = jnp.where(upd2, ov, mkvs[j])
                    mkis[j] = jnp.where(upd2, i, mkis[j])
                bo_v[sl] = cur
                bk_v[sl] = curk
            return tuple(mkvs), tuple(mkis)

        mkvs, mkis = lax.fori_loop(
            0, _NCHUNK // 2, body,
            ((jnp.full((16,), -1.0, f32),) * 4, (i32z,) * 4))
        for j in range(4):
            mkv = mkvs[j]
            flat = mkis[j] * 16 + lane
            m = mkv[0]
            fi = flat[0]
            for t in range(1, 16):
                vt = mkv[t]
                ft = flat[t]
                take = (vt > m) | ((vt == m) & (ft < fi))
                m = jnp.where(take, vt, m)
                fi = jnp.where(take, ft, fi)
            pfeo.append(fi)

    pfeo_b = [lax.broadcast(p, (16,)) for p in pfeo]
    lv = lab_v[...]
    lab_b = [lax.broadcast(lv[k], (16,)) for k in range(_K)]

    def body3(i2, _):
        for u in range(2):
            i = i2 * 2 + u
            sl = pl.ds(i * 16, 16)
            bk = bk_v[sl]
            bo = bo_v[sl]
            flat = lax.broadcast(i * 16, (16,)) + lane
            for k in range(_K):
                hit = flat == pfeo_b[k]
                bk = jnp.where(hit, k, bk)
                bo = jnp.where(hit, 1.0, bo)
            lab = jnp.zeros((16,), i32)
            for k in range(_K):
                lab = jnp.where(bk == k, lab_b[k], lab)
            lo_v[sl] = jnp.where(bo < _THRESHOLD, 0, lab)
            bk_v[sl] = bk
        return 0

    lax.fori_loop(0, _NCHUNK // 2, body3, 0)
    pltpu.sync_copy(lo_v, label_out.at[wid])
    pltpu.sync_copy(bk_v, obj_out.at[wid])


def _make_sc_match():
    mesh = plsc.VectorSubcoreMesh(core_axis_name="c", subcore_axis_name="s")
    f32 = jnp.float32
    i32 = jnp.int32
    return functools.partial(
        pl.kernel,
        out_type=[jax.ShapeDtypeStruct((_B, _PP), i32),
                  jax.ShapeDtypeStruct((_B, _PP), i32)],
        mesh=mesh,
        scratch_types=[
            pltpu.VMEM((_PP,), f32), pltpu.VMEM((_PP,), f32),
            pltpu.VMEM((_PP,), f32), pltpu.VMEM((_PP,), f32),
            pltpu.VMEM((_PP,), f32),
            pltpu.VMEM((64,), f32), pltpu.VMEM((16,), i32),
            pltpu.VMEM((_PP,), f32), pltpu.VMEM((_PP,), i32),
            pltpu.VMEM((_PP,), i32),
        ],
    )(_sc_match_body)


def _lse_body(scores1_ref, scores2_ref, lse1_ref, lse2_ref):
    s1 = scores1_ref[0]
    mx1 = jnp.max(s1, axis=0, keepdims=True)
    lse1_ref[0] = jnp.log(jnp.sum(jnp.exp(s1 - mx1), axis=0, keepdims=True)) + mx1
    s2 = scores2_ref[0]
    mx2 = jnp.max(s2, axis=0, keepdims=True)
    lse2_ref[0] = jnp.log(jnp.sum(jnp.exp(s2 - mx2), axis=0, keepdims=True)) + mx2


def _ce_body(priors_ref, boxes_ref, label_ref, obj_ref,
             locs1_ref, scores1_ref, locs2_ref, scores2_ref,
             lse1_ref, lse2_ref, cn1_ref, cn2_ref, part_ref):
    f32 = jnp.float32
    K, P, C = _K, _P, _C

    pcx = priors_ref[0:1, :]
    pcy = priors_ref[1:2, :]
    pw = priors_ref[2:3, :]
    ph = priors_ref[3:4, :]

    b = boxes_ref[0]
    bx0 = b[:, 0:1]
    by0 = b[:, 1:2]
    bx1 = b[:, 2:3]
    by1 = b[:, 3:4]

    label = label_ref[0]
    obj = obj_ref[0]
    pos = label > 0
    posf = pos.astype(f32)
    npos = jnp.sum(posf)

    kiota = lax.broadcasted_iota(jnp.int32, (K, P), 0)
    onehotf = (obj == kiota).astype(f32)
    gx0 = jnp.sum(onehotf * bx0, axis=0, keepdims=True)
    gy0 = jnp.sum(onehotf * by0, axis=0, keepdims=True)
    gx1 = jnp.sum(onehotf * bx1, axis=0, keepdims=True)
    gy1 = jnp.sum(onehotf * by1, axis=0, keepdims=True)
    cx = (gx0 + gx1) * 0.5
    cy = (gy0 + gy1) * 0.5
    w = gx1 - gx0
    h = gy1 - gy0
    tl0 = (cx - pcx) / (pw / 10.0)
    tl1 = (cy - pcy) / (ph / 10.0)
    tl2 = jnp.log(w / pw) * 5.0
    tl3 = jnp.log(h / ph) * 5.0

    ciota = lax.broadcasted_iota(jnp.int32, (C, P), 0)

    def branch(locs_ref, scores_ref, lse_ref, cn_ref):
        loc_abs = (jnp.abs(locs_ref[0, 0:1, :] - tl0)
                   + jnp.abs(locs_ref[0, 1:2, :] - tl1)
                   + jnp.abs(locs_ref[0, 2:3, :] - tl2)
                   + jnp.abs(locs_ref[0, 3:4, :] - tl3))
        loc_sum = jnp.sum(loc_abs * posf)
        s = scores_ref[0]
        lse = lse_ref[0]
        strue = jnp.sum(jnp.where(ciota == label, s, 0.0), axis=0, keepdims=True)
        ce = lse - strue
        cep = jnp.sum(ce * posf)
        cn = jnp.maximum(jnp.where(pos, 0.0, ce), 0.0)
        cn_ref[0] = cn
        return loc_sum, cep

    l1, c1 = branch(locs1_ref, scores1_ref, lse1_ref, cn1_ref)
    l2, c2 = branch(locs2_ref, scores2_ref, lse2_ref, cn2_ref)

    lane = lax.broadcasted_iota(jnp.int32, (1, 128), 1)
    row = (npos * (lane == 0).astype(f32)
           + l1 * (lane == 1).astype(f32)
           + l2 * (lane == 2).astype(f32)
           + c1 * (lane == 3).astype(f32)
           + c2 * (lane == 4).astype(f32))
    part_ref[0] = row


def _hardneg_body(cn1_ref, cn2_ref, part_ref, out1_ref, out2_ref):
    f32 = jnp.float32
    parts = part_ref[...]
    npos = parts[:, 0:1]
    l1_tot = jnp.sum(parts[:, 1:2])
    l2_tot = jnp.sum(parts[:, 2:3])
    c1_tot = jnp.sum(parts[:, 3:4])
    c2_tot = jnp.sum(parts[:, 4:5])
    np_tot = jnp.sum(npos)
    m = npos * float(_NEG_POS_RATIO)

    def topm_sum(v):
        t = jnp.zeros((_B, 1), jnp.int32)
        for bit in range(30, -1, -1):
            cand = t | (1 << bit)
            tf = lax.bitcast_convert_type(cand, f32)
            cnt = jnp.sum((v >= tf).astype(f32), axis=1, keepdims=True)
            t = jnp.where(cnt >= m, cand, t)
        tf = lax.bitcast_convert_type(t, f32)
        gtf = (v > tf).astype(f32)
        cnt_gt = jnp.sum(gtf, axis=1, keepdims=True)
        hard = jnp.sum(v * gtf, axis=1, keepdims=True) + (m - cnt_gt) * tf
        return jnp.sum(hard)

    h1 = topm_sum(cn1_ref[...])
    h2 = topm_sum(cn2_ref[...])
    o1 = (h1 + c1_tot) / np_tot + _ALPHA * l1_tot / (np_tot * 4.0)
    o2 = (h2 + c2_tot) / np_tot + _ALPHA * l2_tot / (np_tot * 4.0)
    out1_ref[...] = o1.reshape(1, 1)
    out2_ref[...] = o2.reshape(1, 1)


def kernel(predicted_locs1, predicted_scores1, predicted_locs2,
           predicted_scores2, boxes, labels, priors_cxcy):
    B, P, C, K = _B, _P, _C, _K
    priors_t = priors_cxcy.T
    locs1_t = jnp.transpose(predicted_locs1, (0, 2, 1))
    locs2_t = jnp.transpose(predicted_locs2, (0, 2, 1))
    scores1_t = jnp.transpose(predicted_scores1, (0, 2, 1))
    scores2_t = jnp.transpose(predicted_scores2, (0, 2, 1))

    pcx, pcy, pw, ph = (priors_cxcy[:, 0], priors_cxcy[:, 1],
                        priors_cxcy[:, 2], priors_cxcy[:, 3])
    pad = _PP - P
    px0 = jnp.pad(pcx - pw * 0.5, (0, pad))
    py0 = jnp.pad(pcy - ph * 0.5, (0, pad))
    px1 = jnp.pad(pcx + pw * 0.5, (0, pad))
    py1 = jnp.pad(pcy + ph * 0.5, (0, pad))
    areap = (px1 - px0) * (py1 - py0)
    boxes_flat = jnp.pad(boxes, ((0, 0), (0, 16 - K), (0, 0))).reshape(B * 64)
    labels_flat = jnp.pad(labels.astype(jnp.int32),
                          ((0, 0), (0, 16 - K))).reshape(B * 16)
    label_pp, obj_pp = _make_sc_match()(
        px0, py0, px1, py1, areap, boxes_flat, labels_flat)
    label_bp = label_pp[:, :P].reshape(B, 1, P)
    obj_bp = obj_pp[:, :P].reshape(B, 1, P)

    lse1, lse2 = pl.pallas_call(
        _lse_body,
        grid=(B,),
        in_specs=[
            pl.BlockSpec((1, C, P), lambda i: (i, 0, 0)),
            pl.BlockSpec((1, C, P), lambda i: (i, 0, 0)),
        ],
        out_specs=[
            pl.BlockSpec((1, 1, P), lambda i: (i, 0, 0)),
            pl.BlockSpec((1, 1, P), lambda i: (i, 0, 0)),
        ],
        out_shape=[
            jax.ShapeDtypeStruct((B, 1, P), jnp.float32),
            jax.ShapeDtypeStruct((B, 1, P), jnp.float32),
        ],
    )(scores1_t, scores2_t)

    cn1, cn2, part = pl.pallas_call(
        _ce_body,
        grid=(B,),
        in_specs=[
            pl.BlockSpec((4, P), lambda i: (0, 0)),
            pl.BlockSpec((1, K, 4), lambda i: (i, 0, 0)),
            pl.BlockSpec((1, 1, P), lambda i: (i, 0, 0)),
            pl.BlockSpec((1, 1, P), lambda i: (i, 0, 0)),
            pl.BlockSpec((1, 4, P), lambda i: (i, 0, 0)),
            pl.BlockSpec((1, C, P), lambda i: (i, 0, 0)),
            pl.BlockSpec((1, 4, P), lambda i: (i, 0, 0)),
            pl.BlockSpec((1, C, P), lambda i: (i, 0, 0)),
            pl.BlockSpec((1, 1, P), lambda i: (i, 0, 0)),
            pl.BlockSpec((1, 1, P), lambda i: (i, 0, 0)),
        ],
        out_specs=[
            pl.BlockSpec((1, 1, P), lambda i: (i, 0, 0)),
            pl.BlockSpec((1, 1, P), lambda i: (i, 0, 0)),
            pl.BlockSpec((1, 1, 128), lambda i: (i, 0, 0)),
        ],
        out_shape=[
            jax.ShapeDtypeStruct((B, 1, P), jnp.float32),
            jax.ShapeDtypeStruct((B, 1, P), jnp.float32),
            jax.ShapeDtypeStruct((B, 1, 128), jnp.float32),
        ],
    )(priors_t, boxes, label_bp, obj_bp, locs1_t, scores1_t, locs2_t, scores2_t,
      lse1, lse2)

    o1, o2 = pl.pallas_call(
        _hardneg_body,
        in_specs=[
            pl.BlockSpec((B, P), lambda: (0, 0)),
            pl.BlockSpec((B, P), lambda: (0, 0)),
            pl.BlockSpec((B, 128), lambda: (0, 0)),
        ],
        out_specs=[
            pl.BlockSpec((1, 1), lambda: (0, 0)),
            pl.BlockSpec((1, 1), lambda: (0, 0)),
        ],
        out_shape=[
            jax.ShapeDtypeStruct((1, 1), jnp.float32),
            jax.ShapeDtypeStruct((1, 1), jnp.float32),
        ],
    )(cn1.reshape(B, P), cn2.reshape(B, P), part.reshape(B, 128))

    return (o1.reshape(()), o2.reshape(()))
